# Initial kernel scaffold; baseline (speedup 1.0000x reference)
import functools

import jax
import jax.numpy as jnp
from jax import lax
from jax.experimental import pallas as pl
from jax.experimental.pallas import tpu as pltpu
from jax.experimental.pallas import tpu_sc as plsc

_NEG = -1e30


def _sc_gather(table, idx3, n_out_rows):
    nw, chunks, chunk = idx3.shape
    d = table.shape[1]
    rows_per_w = chunks * chunk
    mesh = plsc.VectorSubcoreMesh(core_axis_name="c", subcore_axis_name="s")

    @functools.partial(
        pl.kernel,
        mesh=mesh,
        out_type=jax.ShapeDtypeStruct((n_out_rows, d), table.dtype),
        scratch_types=[
            pltpu.VMEM((chunks, chunk), jnp.int32),
            pltpu.VMEM((chunk, d), table.dtype),
            pltpu.SemaphoreType.DMA,
        ],
    )
    def k(table_hbm, idx_hbm, out_hbm, idx_v, rows_v, sem):
        wid = lax.axis_index("s") * 2 + lax.axis_index("c")
        pltpu.sync_copy(idx_hbm.at[wid], idx_v)
        base = wid * rows_per_w
        for c in range(chunks):
            pltpu.async_copy(table_hbm.at[idx_v.at[c]], rows_v, sem).wait()
            pltpu.sync_copy(rows_v, out_hbm.at[pl.ds(base + c * chunk, chunk)])

    return k(table, idx3)


def kernel(x, Wr, router_bias, Wg, Wu, Wd, Wsg, Wsu, Wsd):
    Bb, Tt, D = x.shape
    N = Bb * Tt
    E, _, H = Wg.shape
    S, _, HS = Wsg.shape
    K = 2
    M = 128
    NB = (K * N) // M + E
    NSLOT = NB * M
    TB = 256
    NW = 32

    flat = x.reshape(N, D)
    f32 = jnp.float32

    wrt = jnp.zeros((D, 128), f32).at[:, :E].set(Wr.T)
    biasp = jnp.full((1, 128), _NEG, f32).at[0, :E].set(router_bias)

    def router_body(x_ref, wrt_ref, bias_ref, idx_ref, w_ref):
        xb = x_ref[...]
        logits = jnp.dot(xb, wrt_ref[...], preferred_element_type=f32)
        lane = lax.broadcasted_iota(jnp.int32, logits.shape, 1)
        valid = lane < E
        lm = jnp.where(valid, logits, _NEG)
        mx = jnp.max(lm, axis=1, keepdims=True)
        p = jnp.exp(lm - mx)
        scores = p / jnp.sum(p, axis=1, keepdims=True)
        sel = lm + bias_ref[...]
        m0 = jnp.max(sel, axis=1, keepdims=True)
        i0 = jnp.min(jnp.where(sel == m0, lane, 1000), axis=1, keepdims=True)
        pick0 = lane == i0
        sel2 = jnp.where(pick0, _NEG, sel)
        m1 = jnp.max(sel2, axis=1, keepdims=True)
        i1 = jnp.min(jnp.where(sel2 == m1, lane, 1000), axis=1, keepdims=True)
        pick1 = lane == i1
        w0 = jnp.sum(jnp.where(pick0, scores, 0.0), axis=1, keepdims=True)
        w1 = jnp.sum(jnp.where(pick1, scores, 0.0), axis=1, keepdims=True)
        tot = w0 + w1
        idx_ref[...] = jnp.where(lane == 0, i0, jnp.where(lane == 1, i1, 0))
        w_ref[...] = jnp.where(
            lane == 0, w0 / tot, jnp.where(lane == 1, w1 / tot, 0.0))

    idx_out, w_out = pl.pallas_call(
        router_body,
        grid=(N // TB,),
        in_specs=[
            pl.BlockSpec((TB, D), lambda i: (i, 0)),
            pl.BlockSpec((D, 128), lambda i: (0, 0)),
            pl.BlockSpec((1, 128), lambda i: (0, 0)),
        ],
        out_specs=[
            pl.BlockSpec((TB, 128), lambda i: (i, 0)),
            pl.BlockSpec((TB, 128), lambda i: (i, 0)),
        ],
        out_shape=[
            jax.ShapeDtypeStruct((N, 128), jnp.int32),
            jax.ShapeDtypeStruct((N, 128), f32),
        ],
    )(flat, wrt, biasp)

    e_flat = idx_out[:, :K].reshape(-1)
    onehot = (e_flat[:, None] == jnp.arange(E)[None, :]).astype(jnp.int32)
    cum = jnp.cumsum(onehot, axis=0)
    counts = cum[-1]
    rank = jnp.sum(onehot * cum, axis=1) - 1
    bc = (counts + M - 1) // M
    blk_start = jnp.concatenate([jnp.zeros((1,), jnp.int32),
                                 jnp.cumsum(bc)[:-1].astype(jnp.int32)])
    slot = (blk_start * M)[e_flat] + rank
    slot_token = jnp.zeros((NSLOT,), jnp.int32).at[slot].set(
        jnp.arange(N * K, dtype=jnp.int32) // K)
    pos = slot.reshape(N, K)
    num_active = jnp.sum(bc).astype(jnp.int32)
    barange = jnp.arange(NB, dtype=jnp.int32)
    owner = jnp.sum(
        (blk_start[None, :] <= barange[:, None]).astype(jnp.int32), axis=1) - 1
    last_e = jnp.max(jnp.where(counts > 0, jnp.arange(E), 0)).astype(jnp.int32)
    block_expert = jnp.where(barange < num_active, owner, last_e).astype(
        jnp.int32)
    na_arr = num_active.reshape(1)

    xd = _sc_gather(flat, slot_token.reshape(NW, NSLOT // (NW * 40), 40),
                    NSLOT)

    def grouped_body(be_ref, na_ref, xd_ref, wg_ref, wu_ref, wd_ref, yw_ref):
        b = pl.program_id(0)

        @pl.when(b < na_ref[0])
        def _():
            xb = xd_ref[...]
            g = jnp.dot(xb, wg_ref[0], preferred_element_type=f32)
            u = jnp.dot(xb, wu_ref[0], preferred_element_type=f32)
            h = g * jax.nn.sigmoid(g) * u
            yw_ref[...] = jnp.dot(h, wd_ref[0], preferred_element_type=f32)

    grid_spec = pltpu.PrefetchScalarGridSpec(
        num_scalar_prefetch=2,
        grid=(NB,),
        in_specs=[
            pl.BlockSpec((M, D), lambda b, be, na: (b, 0)),
            pl.BlockSpec((1, D, H), lambda b, be, na: (be[b], 0, 0)),
            pl.BlockSpec((1, D, H), lambda b, be, na: (be[b], 0, 0)),
            pl.BlockSpec((1, H, D), lambda b, be, na: (be[b], 0, 0)),
        ],
        out_specs=pl.BlockSpec((M, D), lambda b, be, na: (b, 0)),
    )
    yw = pl.pallas_call(
        grouped_body,
        grid_spec=grid_spec,
        out_shape=jax.ShapeDtypeStruct((NSLOT, D), f32),
    )(block_expert, na_arr, xd, Wg, Wu, Wd)

    pos_all = jnp.concatenate([pos[:, 0], pos[:, 1]]).astype(jnp.int32)
    yg = _sc_gather(yw, pos_all.reshape(NW, (N * K) // (NW * 32), 32), N * K)

    def shared_body(x_ref, wsg_ref, wsu_ref, wsd_ref, y0_ref, y1_ref, w_ref,
                    o_ref):
        w0 = w_ref[:, 0:1]
        w1 = w_ref[:, 1:2]
        acc = y0_ref[...] * w0 + y1_ref[...] * w1
        xb = x_ref[...]
        for s in range(S):
            g = jnp.dot(xb, wsg_ref[s], preferred_element_type=f32)
            u = jnp.dot(xb, wsu_ref[s], preferred_element_type=f32)
            h = g * jax.nn.sigmoid(g) * u
            acc = acc + jnp.dot(h, wsd_ref[s], preferred_element_type=f32)
        o_ref[...] = acc

    nblk = N // TB
    out = pl.pallas_call(
        shared_body,
        grid=(nblk,),
        in_specs=[
            pl.BlockSpec((TB, D), lambda i: (i, 0)),
            pl.BlockSpec((S, D, HS), lambda i: (0, 0, 0)),
            pl.BlockSpec((S, D, HS), lambda i: (0, 0, 0)),
            pl.BlockSpec((S, HS, D), lambda i: (0, 0, 0)),
            pl.BlockSpec((TB, D), lambda i: (i, 0)),
            pl.BlockSpec((TB, D), lambda i, _n=nblk: (i + _n, 0)),
            pl.BlockSpec((TB, 128), lambda i: (i, 0)),
        ],
        out_specs=pl.BlockSpec((TB, D), lambda i: (i, 0)),
        out_shape=jax.ShapeDtypeStruct((N, D), f32),
    )(flat, Wsg, Wsu, Wsd, yg, yg, w_out)

    return out.reshape(Bb, Tt, D)


# speedup vs baseline: 1.0765x; 1.0765x over previous
import functools

import jax
import jax.numpy as jnp
from jax import lax
from jax.experimental import pallas as pl
from jax.experimental.pallas import tpu as pltpu
from jax.experimental.pallas import tpu_sc as plsc

_NEG = -1e30


def _sc_gather(table, idx3, n_out_rows):
    nw, chunks, chunk = idx3.shape
    d = table.shape[1]
    rows_per_w = chunks * chunk
    mesh = plsc.VectorSubcoreMesh(core_axis_name="c", subcore_axis_name="s")

    @functools.partial(
        pl.kernel,
        mesh=mesh,
        out_type=jax.ShapeDtypeStruct((n_out_rows, d), table.dtype),
        scratch_types=[
            pltpu.VMEM((chunks, chunk), jnp.int32),
            pltpu.VMEM((chunk, d), table.dtype),
            pltpu.SemaphoreType.DMA,
        ],
    )
    def k(table_hbm, idx_hbm, out_hbm, idx_v, rows_v, sem):
        wid = lax.axis_index("s") * 2 + lax.axis_index("c")
        pltpu.sync_copy(idx_hbm.at[wid], idx_v)
        base = wid * rows_per_w
        for c in range(chunks):
            pltpu.async_copy(table_hbm.at[idx_v.at[c]], rows_v, sem).wait()
            pltpu.sync_copy(rows_v, out_hbm.at[pl.ds(base + c * chunk, chunk)])

    return k(table, idx3)


def kernel(x, Wr, router_bias, Wg, Wu, Wd, Wsg, Wsu, Wsd):
    Bb, Tt, D = x.shape
    N = Bb * Tt
    E, _, H = Wg.shape
    S, _, HS = Wsg.shape
    K = 2
    M = 128
    NB = (K * N) // M + E
    NSLOT = NB * M
    TB = 256
    NW = 32

    flat = x.reshape(N, D)
    f32 = jnp.float32

    wrt = jnp.zeros((D, 128), f32).at[:, :E].set(Wr.T)
    biasp = jnp.full((1, 128), _NEG, f32).at[0, :E].set(router_bias)

    def router_body(x_ref, wrt_ref, bias_ref, idx_ref, w_ref):
        xb = x_ref[...]
        logits = jnp.dot(xb, wrt_ref[...], preferred_element_type=f32)
        lane = lax.broadcasted_iota(jnp.int32, logits.shape, 1)
        valid = lane < E
        lm = jnp.where(valid, logits, _NEG)
        mx = jnp.max(lm, axis=1, keepdims=True)
        p = jnp.exp(lm - mx)
        scores = p / jnp.sum(p, axis=1, keepdims=True)
        sel = lm + bias_ref[...]
        m0 = jnp.max(sel, axis=1, keepdims=True)
        i0 = jnp.min(jnp.where(sel == m0, lane, 1000), axis=1, keepdims=True)
        pick0 = lane == i0
        sel2 = jnp.where(pick0, _NEG, sel)
        m1 = jnp.max(sel2, axis=1, keepdims=True)
        i1 = jnp.min(jnp.where(sel2 == m1, lane, 1000), axis=1, keepdims=True)
        pick1 = lane == i1
        w0 = jnp.sum(jnp.where(pick0, scores, 0.0), axis=1, keepdims=True)
        w1 = jnp.sum(jnp.where(pick1, scores, 0.0), axis=1, keepdims=True)
        tot = w0 + w1
        idx_ref[...] = jnp.where(lane == 0, i0, jnp.where(lane == 1, i1, 0))
        w_ref[...] = jnp.where(
            lane == 0, w0 / tot, jnp.where(lane == 1, w1 / tot, 0.0))

    idx_out, w_out = pl.pallas_call(
        router_body,
        grid=(N // TB,),
        in_specs=[
            pl.BlockSpec((TB, D), lambda i: (i, 0)),
            pl.BlockSpec((D, 128), lambda i: (0, 0)),
            pl.BlockSpec((1, 128), lambda i: (0, 0)),
        ],
        out_specs=[
            pl.BlockSpec((TB, 128), lambda i: (i, 0)),
            pl.BlockSpec((TB, 128), lambda i: (i, 0)),
        ],
        out_shape=[
            jax.ShapeDtypeStruct((N, 128), jnp.int32),
            jax.ShapeDtypeStruct((N, 128), f32),
        ],
    )(flat, wrt, biasp)

    e_flat = idx_out[:, :K].reshape(-1)
    onehot = (e_flat[:, None] == jnp.arange(E)[None, :]).astype(jnp.int32)
    cum = jnp.cumsum(onehot, axis=0)
    counts = cum[-1]
    rank = jnp.sum(onehot * cum, axis=1) - 1
    bc = (counts + M - 1) // M
    blk_start = jnp.concatenate([jnp.zeros((1,), jnp.int32),
                                 jnp.cumsum(bc)[:-1].astype(jnp.int32)])
    slot = (blk_start * M)[e_flat] + rank
    slot_token = jnp.zeros((NSLOT,), jnp.int32).at[slot].set(
        jnp.arange(N * K, dtype=jnp.int32) // K)
    pos = slot.reshape(N, K)
    num_active = jnp.sum(bc).astype(jnp.int32)
    barange = jnp.arange(NB, dtype=jnp.int32)
    owner = jnp.sum(
        (blk_start[None, :] <= barange[:, None]).astype(jnp.int32), axis=1) - 1
    last_e = jnp.max(jnp.where(counts > 0, jnp.arange(E), 0)).astype(jnp.int32)
    block_expert = jnp.where(barange < num_active, owner, last_e).astype(
        jnp.int32)
    na_arr = num_active.reshape(1)

    xd = _sc_gather(flat, slot_token.reshape(NW, NSLOT // (NW * 40), 40),
                    NSLOT)

    def grouped_body(be_ref, na_ref, xd_ref, wg_ref, wu_ref, wd_ref, yw_ref):
        b = pl.program_id(0)

        @pl.when(b < na_ref[0])
        def _():
            xb = xd_ref[...]
            g = jnp.dot(xb, wg_ref[0], preferred_element_type=f32)
            u = jnp.dot(xb, wu_ref[0], preferred_element_type=f32)
            h = g * jax.nn.sigmoid(g) * u
            yw_ref[...] = jnp.dot(h, wd_ref[0], preferred_element_type=f32)

    grid_spec = pltpu.PrefetchScalarGridSpec(
        num_scalar_prefetch=2,
        grid=(NB,),
        in_specs=[
            pl.BlockSpec((M, D), lambda b, be, na: (b, 0)),
            pl.BlockSpec((1, D, H), lambda b, be, na: (be[b], 0, 0)),
            pl.BlockSpec((1, D, H), lambda b, be, na: (be[b], 0, 0)),
            pl.BlockSpec((1, H, D), lambda b, be, na: (be[b], 0, 0)),
        ],
        out_specs=pl.BlockSpec((M, D), lambda b, be, na: (b, 0)),
    )
    yw = pl.pallas_call(
        grouped_body,
        grid_spec=grid_spec,
        out_shape=jax.ShapeDtypeStruct((NSLOT, D), f32),
    )(block_expert, na_arr, xd, Wg, Wu, Wd)

    pos_all = jnp.concatenate([pos[:, 0], pos[:, 1]]).astype(jnp.int32)
    yg = _sc_gather(yw, pos_all.reshape(NW, (N * K) // (NW * 32), 32), N * K)

    CH = 512
    HC = HS // CH
    NJ = S * HC

    def shared_body(x_ref, wsg_ref, wsu_ref, wsd_ref, y0_ref, y1_ref, w_ref,
                    o_ref):
        j = pl.program_id(1)
        xb = x_ref[...]
        g = jnp.dot(xb, wsg_ref[0], preferred_element_type=f32)
        u = jnp.dot(xb, wsu_ref[0], preferred_element_type=f32)
        h = g * jax.nn.sigmoid(g) * u
        part = jnp.dot(h, wsd_ref[0], preferred_element_type=f32)

        @pl.when(j == 0)
        def _():
            o_ref[...] = (part + y0_ref[...] * w_ref[:, 0:1]
                          + y1_ref[...] * w_ref[:, 1:2])

        @pl.when(j > 0)
        def _():
            o_ref[...] += part

    nblk = N // TB
    out = pl.pallas_call(
        shared_body,
        grid=(nblk, NJ),
        in_specs=[
            pl.BlockSpec((TB, D), lambda i, j: (i, 0)),
            pl.BlockSpec((1, D, CH), lambda i, j, _h=HC: (j // _h, 0, j % _h)),
            pl.BlockSpec((1, D, CH), lambda i, j, _h=HC: (j // _h, 0, j % _h)),
            pl.BlockSpec((1, CH, D), lambda i, j, _h=HC: (j // _h, j % _h, 0)),
            pl.BlockSpec((TB, D), lambda i, j: (i, 0)),
            pl.BlockSpec((TB, D), lambda i, j, _n=nblk: (i + _n, 0)),
            pl.BlockSpec((TB, 128), lambda i, j: (i, 0)),
        ],
        out_specs=pl.BlockSpec((TB, D), lambda i, j: (i, 0)),
        out_shape=jax.ShapeDtypeStruct((N, D), f32),
    )(flat, Wsg, Wsu, Wsd, yg, yg, w_out)

    return out.reshape(Bb, Tt, D)
